# Initial kernel scaffold; baseline (speedup 1.0000x reference)
import jax
import jax.numpy as jnp
from jax.experimental import pallas as pl


def kernel(logits, labels):
    raise NotImplementedError("write your pallas kernel here")


# speedup vs baseline: 8.1737x; 8.1737x over previous
import functools

import jax
import jax.numpy as jnp
from jax import lax
from jax.experimental import pallas as pl
from jax.experimental.pallas import tpu as pltpu
from jax.experimental.pallas import tpu_sc as plsc

B = 16
N = 512 * 512
LANES = 16
SHIFT = 18
NB = 1 << (32 - SHIFT)
CHUNK = 2048
NCHUNK = N // CHUNK
NVEC = CHUNK // LANES
NBVEC = NB // LANES


def _body(logits_hbm, labels_hbm, out_hbm, lbuf, ybuf, hn, hp, hs, obuf):
    cid = lax.axis_index("c")
    sid = lax.axis_index("s")
    wid = cid * 16 + sid

    @pl.when(wid < B)
    def _():
        img = wid
        zeros = jnp.zeros((LANES,), jnp.float32)
        ones = jnp.ones((LANES,), jnp.float32)

        def _zero(i, _):
            hn[pl.ds(i * LANES, LANES)] = zeros
            hp[pl.ds(i * LANES, LANES)] = zeros
            hs[pl.ds(i * LANES, LANES)] = zeros
            return 0

        lax.fori_loop(0, NBVEC, _zero, 0)

        def _chunk(c, _):
            base = img * N + c * CHUNK
            pltpu.sync_copy(logits_hbm.at[pl.ds(base, CHUNK)], lbuf)
            pltpu.sync_copy(labels_hbm.at[pl.ds(base, CHUNK)], ybuf)

            def _vec(i, _):
                l = lbuf[pl.ds(i * LANES, LANES)]
                y = ybuf[pl.ds(i * LANES, LANES)]
                yf = y.astype(jnp.float32)
                e = 1.0 - l * (2.0 * yf - 1.0)
                act = jnp.where(e > 0.0, e + 1.0, jnp.exp(e))
                bits = lax.bitcast_convert_type(e, jnp.int32)
                xm = (bits >> 31) | jnp.int32(-(2**31))
                key = bits ^ xm
                bin_ = lax.shift_right_logical(key, SHIFT)
                plsc.addupdate_scatter(hn, [bin_], ones)
                plsc.addupdate_scatter(hp, [bin_], yf)
                plsc.addupdate_scatter(hs, [bin_], act)
                return 0

            lax.fori_loop(0, NVEC, _vec, 0)
            return 0

        lax.fori_loop(0, NCHUNK, _chunk, 0)

        def _gsum(i, g):
            return g + jnp.sum(hp[pl.ds(i * LANES, LANES)])

        g = lax.fori_loop(0, NBVEC, _gsum, jnp.float32(0.0))
        totn = jnp.float32(N)

        def _scan(i, carry):
            accn, accp, accl = carry
            n = hn[pl.ds(i * LANES, LANES)]
            p = hp[pl.ds(i * LANES, LANES)]
            s = hs[pl.ds(i * LANES, LANES)]
            cn = plsc.cumsum(n) + accn
            cp = plsc.cumsum(p) + accp
            n_excl = totn - cn
            p_excl = g - cp
            n_incl = n_excl + n
            p_incl = p_excl + p
            jb = 1.0 - (g - p_incl) / jnp.maximum(g + n_incl - p_incl, 1.0)
            ja = 1.0 - (g - p_excl) / jnp.maximum(g + n_excl - p_excl, 1.0)
            accl = accl + s * (jb - ja) / jnp.maximum(n, 1.0)
            return (accn + jnp.sum(n), accp + jnp.sum(p), accl)

        _, _, accl = lax.fori_loop(
            0, NBVEC, _scan,
            (jnp.float32(0.0), jnp.float32(0.0), jnp.zeros((LANES,), jnp.float32)))

        loss = jnp.sum(accl)
        obuf[...] = jnp.broadcast_to(loss, (LANES,))
        pltpu.sync_copy(obuf, out_hbm.at[img])


@jax.jit
def _lovasz_sc(logits_flat, labels_flat):
    mesh = plsc.VectorSubcoreMesh(core_axis_name="c", subcore_axis_name="s")
    return pl.kernel(
        _body,
        out_type=jax.ShapeDtypeStruct((B, LANES), jnp.float32),
        mesh=mesh,
        compiler_params=pltpu.CompilerParams(needs_layout_passes=False),
        scratch_types=[
            pltpu.VMEM((CHUNK,), jnp.float32),
            pltpu.VMEM((CHUNK,), jnp.int32),
            pltpu.VMEM((NB,), jnp.float32),
            pltpu.VMEM((NB,), jnp.float32),
            pltpu.VMEM((NB,), jnp.float32),
            pltpu.VMEM((LANES,), jnp.float32),
        ],
    )(logits_flat, labels_flat)


def kernel(logits, labels):
    out = _lovasz_sc(logits.reshape(-1), labels.reshape(-1))
    return jnp.mean(out[:, 0])


# speedup vs baseline: 11.6036x; 1.4196x over previous
import functools

import jax
import jax.numpy as jnp
from jax import lax
from jax.experimental import pallas as pl
from jax.experimental.pallas import tpu as pltpu
from jax.experimental.pallas import tpu_sc as plsc

B = 16
N = 512 * 512
LANES = 16
SHIFT = 18
NB = 1 << (32 - SHIFT)
CHUNK = 2048
NCHUNK = N // CHUNK
NVEC = CHUNK // LANES
UNROLL = 4
NBVEC = NB // LANES


def _body(logits_hbm, labels_hbm, out_hbm, lbuf, ybuf, hn, hp, hs, obuf,
          sem0, sem1):
    cid = lax.axis_index("c")
    sid = lax.axis_index("s")
    wid = cid * 16 + sid

    @pl.when(wid < B)
    def _():
        img = wid
        zeros = jnp.zeros((LANES,), jnp.float32)
        ones = jnp.ones((LANES,), jnp.float32)

        def _zero(i, _):
            hn[pl.ds(i * LANES, LANES)] = zeros
            hp[pl.ds(i * LANES, LANES)] = zeros
            hs[pl.ds(i * LANES, LANES)] = zeros
            return 0

        lax.fori_loop(0, NBVEC, _zero, 0)

        sems = (sem0, sem1)

        def _start(c, slot):
            base = img * N + c * CHUNK
            pltpu.async_copy(logits_hbm.at[pl.ds(base, CHUNK)],
                             lbuf.at[slot], sems[slot])
            pltpu.async_copy(labels_hbm.at[pl.ds(base, CHUNK)],
                             ybuf.at[slot], sems[slot])

        def _drain(slot):
            pltpu.make_async_copy(logits_hbm.at[pl.ds(0, CHUNK)],
                                  lbuf.at[slot], sems[slot]).wait()
            pltpu.make_async_copy(labels_hbm.at[pl.ds(0, CHUNK)],
                                  ybuf.at[slot], sems[slot]).wait()

        def _process(slot):
            def _vec(i, _):
                for u in range(UNROLL):
                    off = (i * UNROLL + u) * LANES
                    l = lbuf[slot, pl.ds(off, LANES)]
                    y = ybuf[slot, pl.ds(off, LANES)]
                    yf = y.astype(jnp.float32)
                    e = 1.0 - l * (2.0 * yf - 1.0)
                    act = jnp.where(e > 0.0, e + 1.0, jnp.exp(e))
                    bits = lax.bitcast_convert_type(e, jnp.int32)
                    xm = (bits >> 31) | jnp.int32(-(2**31))
                    key = bits ^ xm
                    bin_ = lax.shift_right_logical(key, SHIFT)
                    plsc.addupdate_scatter(hn, [bin_], ones)
                    plsc.addupdate_scatter(hp, [bin_], yf)
                    plsc.addupdate_scatter(hs, [bin_], act)
                return 0

            lax.fori_loop(0, NVEC // UNROLL, _vec, 0)

        _start(0, 0)

        def _chunk2(c2, _):
            _drain(0)
            _start(c2 * 2 + 1, 1)
            _process(0)
            _drain(1)

            @pl.when(c2 * 2 + 2 < NCHUNK)
            def _():
                _start(c2 * 2 + 2, 0)

            _process(1)
            return 0

        lax.fori_loop(0, NCHUNK // 2, _chunk2, 0)

        def _gsum(i, g):
            return g + jnp.sum(hp[pl.ds(i * LANES, LANES)])

        g = lax.fori_loop(0, NBVEC, _gsum, jnp.float32(0.0))
        totn = jnp.float32(N)

        def _scan(i, carry):
            accn, accp, accl = carry
            n = hn[pl.ds(i * LANES, LANES)]
            p = hp[pl.ds(i * LANES, LANES)]
            s = hs[pl.ds(i * LANES, LANES)]
            cn = plsc.cumsum(n) + accn
            cp = plsc.cumsum(p) + accp
            n_excl = totn - cn
            p_excl = g - cp
            n_incl = n_excl + n
            p_incl = p_excl + p
            jb = 1.0 - (g - p_incl) / jnp.maximum(g + n_incl - p_incl, 1.0)
            ja = 1.0 - (g - p_excl) / jnp.maximum(g + n_excl - p_excl, 1.0)
            accl = accl + s * (jb - ja) / jnp.maximum(n, 1.0)
            return (accn + jnp.sum(n), accp + jnp.sum(p), accl)

        _, _, accl = lax.fori_loop(
            0, NBVEC, _scan,
            (jnp.float32(0.0), jnp.float32(0.0), jnp.zeros((LANES,), jnp.float32)))

        loss = jnp.sum(accl)
        obuf[...] = jnp.broadcast_to(loss, (LANES,))
        pltpu.sync_copy(obuf, out_hbm.at[img])


@jax.jit
def _lovasz_sc(logits_flat, labels_flat):
    mesh = plsc.VectorSubcoreMesh(core_axis_name="c", subcore_axis_name="s")
    return pl.kernel(
        _body,
        out_type=jax.ShapeDtypeStruct((B, LANES), jnp.float32),
        mesh=mesh,
        compiler_params=pltpu.CompilerParams(needs_layout_passes=False),
        scratch_types=[
            pltpu.VMEM((2, CHUNK), jnp.float32),
            pltpu.VMEM((2, CHUNK), jnp.int32),
            pltpu.VMEM((NB,), jnp.float32),
            pltpu.VMEM((NB,), jnp.float32),
            pltpu.VMEM((NB,), jnp.float32),
            pltpu.VMEM((LANES,), jnp.float32),
            pltpu.SemaphoreType.DMA,
            pltpu.SemaphoreType.DMA,
        ],
    )(logits_flat, labels_flat)


def kernel(logits, labels):
    out = _lovasz_sc(logits.reshape(-1), labels.reshape(-1))
    return jnp.mean(out[:, 0])


# speedup vs baseline: 18.3128x; 1.5782x over previous
import functools

import jax
import jax.numpy as jnp
from jax import lax
from jax.experimental import pallas as pl
from jax.experimental.pallas import tpu as pltpu
from jax.experimental.pallas import tpu_sc as plsc

B = 16
N = 512 * 512
LANES = 16
SHIFT = 18
NB = 1 << (32 - SHIFT)
CHUNK = 2048
NCHUNK = N // CHUNK
NVEC = CHUNK // LANES
NHCHUNK = (N // 2) // CHUNK
UNROLL = 4
NBVEC = NB // LANES


def _body(logits_hbm, labels_hbm, out_hbm, lbuf, ybuf, hn, hp, hs, obuf,
          pbuf, shared, sem0, sem1):
    cid = lax.axis_index("c")
    sid = lax.axis_index("s")
    wid = cid * 16 + sid
    img = wid // 2
    half = wid % 2

    zeros = jnp.zeros((LANES,), jnp.float32)
    ones = jnp.ones((LANES,), jnp.float32)

    def _zero(i, _):
        hn[pl.ds(i * LANES, LANES)] = zeros
        hp[pl.ds(i * LANES, LANES)] = zeros
        hs[pl.ds(i * LANES, LANES)] = zeros
        return 0

    lax.fori_loop(0, NBVEC, _zero, 0)

    sems = (sem0, sem1)

    def _start(c, slot):
        base = img * N + half * (N // 2) + c * CHUNK
        pltpu.async_copy(logits_hbm.at[pl.ds(base, CHUNK)],
                         lbuf.at[slot], sems[slot])
        pltpu.async_copy(labels_hbm.at[pl.ds(base, CHUNK)],
                         ybuf.at[slot], sems[slot])

    def _drain(slot):
        pltpu.make_async_copy(logits_hbm.at[pl.ds(0, CHUNK)],
                              lbuf.at[slot], sems[slot]).wait()
        pltpu.make_async_copy(labels_hbm.at[pl.ds(0, CHUNK)],
                              ybuf.at[slot], sems[slot]).wait()

    def _process(slot):
        def _vec(i, _):
            for u in range(UNROLL):
                off = (i * UNROLL + u) * LANES
                l = lbuf[slot, pl.ds(off, LANES)]
                y = ybuf[slot, pl.ds(off, LANES)]
                yf = y.astype(jnp.float32)
                e = 1.0 - l * (2.0 * yf - 1.0)
                act = jnp.where(e > 0.0, e + 1.0, jnp.exp(e))
                bits = lax.bitcast_convert_type(e, jnp.int32)
                xm = (bits >> 31) | jnp.int32(-(2**31))
                key = bits ^ xm
                bin_ = lax.shift_right_logical(key, SHIFT)
                plsc.addupdate_scatter(hn, [bin_], ones)
                plsc.addupdate_scatter(hp, [bin_], yf)
                plsc.addupdate_scatter(hs, [bin_], act)
            return 0

        lax.fori_loop(0, NVEC // UNROLL, _vec, 0)

    _start(0, 0)

    def _chunk2(c2, _):
        _drain(0)
        _start(c2 * 2 + 1, 1)
        _process(0)
        _drain(1)

        @pl.when(c2 * 2 + 2 < NHCHUNK)
        def _():
            _start(c2 * 2 + 2, 0)

        _process(1)
        return 0

    lax.fori_loop(0, NHCHUNK // 2, _chunk2, 0)

    @pl.when(half == 1)
    def _():
        pltpu.sync_copy(hn, shared.at[pl.ds((sid * 3 + 0) * NB, NB)])
        pltpu.sync_copy(hp, shared.at[pl.ds((sid * 3 + 1) * NB, NB)])
        pltpu.sync_copy(hs, shared.at[pl.ds((sid * 3 + 2) * NB, NB)])

    plsc.subcore_barrier()

    @pl.when(half == 0)
    def _():
        for k, h in enumerate((hn, hp, hs)):
            pltpu.sync_copy(shared.at[pl.ds(((sid + 1) * 3 + k) * NB, NB)], pbuf)

            def _merge(i, _):
                sl = pl.ds(i * LANES, LANES)
                h[sl] = h[sl] + pbuf[sl]
                return 0

            lax.fori_loop(0, NBVEC, _merge, 0)

        def _gsum(i, g):
            return g + jnp.sum(hp[pl.ds(i * LANES, LANES)])

        g = lax.fori_loop(0, NBVEC, _gsum, jnp.float32(0.0))
        totn = jnp.float32(N)

        def _scan(i, carry):
            accn, accp, accl = carry
            n = hn[pl.ds(i * LANES, LANES)]
            p = hp[pl.ds(i * LANES, LANES)]
            s = hs[pl.ds(i * LANES, LANES)]
            cn = plsc.cumsum(n) + accn
            cp = plsc.cumsum(p) + accp
            n_excl = totn - cn
            p_excl = g - cp
            n_incl = n_excl + n
            p_incl = p_excl + p
            jb = 1.0 - (g - p_incl) / jnp.maximum(g + n_incl - p_incl, 1.0)
            ja = 1.0 - (g - p_excl) / jnp.maximum(g + n_excl - p_excl, 1.0)
            accl = accl + s * (jb - ja) / jnp.maximum(n, 1.0)
            return (accn + jnp.sum(n), accp + jnp.sum(p), accl)

        _, _, accl = lax.fori_loop(
            0, NBVEC, _scan,
            (jnp.float32(0.0), jnp.float32(0.0), jnp.zeros((LANES,), jnp.float32)))

        loss = jnp.sum(accl)
        obuf[...] = jnp.broadcast_to(loss, (LANES,))
        pltpu.sync_copy(obuf, out_hbm.at[img])


@jax.jit
def _lovasz_sc(logits_flat, labels_flat):
    mesh = plsc.VectorSubcoreMesh(core_axis_name="c", subcore_axis_name="s")
    return pl.kernel(
        _body,
        out_type=jax.ShapeDtypeStruct((B, LANES), jnp.float32),
        mesh=mesh,
        compiler_params=pltpu.CompilerParams(needs_layout_passes=False),
        scratch_types=[
            pltpu.VMEM((2, CHUNK), jnp.float32),
            pltpu.VMEM((2, CHUNK), jnp.int32),
            pltpu.VMEM((NB,), jnp.float32),
            pltpu.VMEM((NB,), jnp.float32),
            pltpu.VMEM((NB,), jnp.float32),
            pltpu.VMEM((LANES,), jnp.float32),
            pltpu.VMEM((NB,), jnp.float32),
            pltpu.VMEM_SHARED((16 * 3 * NB,), jnp.float32),
            pltpu.SemaphoreType.DMA,
            pltpu.SemaphoreType.DMA,
        ],
    )(logits_flat, labels_flat)


def kernel(logits, labels):
    out = _lovasz_sc(logits.reshape(-1), labels.reshape(-1))
    return jnp.mean(out[:, 0])
